# baseline (speedup 1.0000x reference)
import functools

import jax
import jax.numpy as jnp
from jax import lax
from jax.experimental import pallas as pl
from jax.experimental.pallas import tpu as pltpu
from jax.experimental.pallas import tpu_sc as plsc

_B, _L, _V, _D = 1024, 200, 1_000_000, 64
_DP = 128
_N = _B * _L
_C = 128
_NW = 32
_RPW = _N // _NW
_CPW = _RPW // _C
_BV2 = 16384
_NSB = -(-_V // (2 * _BV2))
_VP = _NSB * _BV2
_NBUF = 4


def _transpose_pack(table_t):
    grid = (_NSB,)

    def body(in1_ref, in2_ref, out_ref):
        out_ref[...] = jnp.concatenate(
            [in1_ref[...].T, in2_ref[...].T], axis=1
        )

    return pl.pallas_call(
        body,
        grid=grid,
        in_specs=[
            pl.BlockSpec((_D, _BV2), lambda i: (0, 2 * i)),
            pl.BlockSpec((_D, _BV2), lambda i: (0, 2 * i + 1)),
        ],
        out_specs=pl.BlockSpec((_BV2, 2 * _D), lambda i: (i, 0)),
        out_shape=jax.ShapeDtypeStruct((_VP, 2 * _D), jnp.float32),
    )(table_t, table_t)


def _make_gather():
    mesh = plsc.VectorSubcoreMesh(core_axis_name="c", subcore_axis_name="s")

    @functools.partial(
        pl.kernel,
        mesh=mesh,
        out_type=jax.ShapeDtypeStruct((_N, _DP), jnp.float32),
        scratch_types=[
            pltpu.VMEM((_RPW,), jnp.int32),
            pltpu.VMEM((_NBUF, _C, _DP), jnp.float32),
            pltpu.SemaphoreType.DMA((_NBUF,)),
            pltpu.SemaphoreType.DMA((_NBUF,)),
        ],
    )
    def gather_kernel(idx_hbm, table_hbm, out_hbm, idx_v, rows_v, gsems, osems):
        wid = lax.axis_index("s") * 2 + lax.axis_index("c")
        rbase = wid * _RPW
        pltpu.sync_copy(idx_hbm.at[pl.ds(rbase, _RPW)], idx_v)

        def fire(j):
            b = j % _NBUF
            idx_slice = idx_v.at[pl.ds(j * _C, _C)]
            pltpu.async_copy(table_hbm.at[idx_slice], rows_v.at[b], gsems.at[b])

        for j in range(_NBUF - 1):
            fire(j)

        def body(j, carry):
            b = j % _NBUF
            nb = (j + _NBUF - 1) % _NBUF

            @pl.when((j >= 1) & (j + _NBUF - 1 < _CPW))
            def _():
                pltpu.make_async_copy(
                    rows_v.at[nb], out_hbm.at[pl.ds(rbase, _C)], osems.at[nb]
                ).wait()

            @pl.when(j + _NBUF - 1 < _CPW)
            def _():
                fire(j + _NBUF - 1)

            pltpu.make_async_copy(
                table_hbm.at[idx_v.at[pl.ds(0, _C)]], rows_v.at[b], gsems.at[b]
            ).wait()
            pltpu.async_copy(
                rows_v.at[b], out_hbm.at[pl.ds(rbase + j * _C, _C)], osems.at[b]
            )
            return carry

        lax.fori_loop(0, _CPW, body, 0)
        for b in range(_NBUF):
            pltpu.make_async_copy(
                rows_v.at[b], out_hbm.at[pl.ds(rbase, _C)], osems.at[b]
            ).wait()

    return gather_kernel


_gather = _make_gather()


def kernel(sequences, offsets, table):
    del offsets
    idx = sequences.reshape(_N).astype(jnp.int32)
    table_p = _transpose_pack(table.T)
    row = ((idx >> 15) << 14) | (idx & (_BV2 - 1))
    pairs = _gather(row, table_p)
    hi = ((idx >> 14) & 1).astype(jnp.bool_)[:, None]
    out = jnp.where(hi, pairs[:, _D:], pairs[:, :_D])
    return out.reshape(_B, _L, _D)


# speedup vs baseline: 1.2699x; 1.2699x over previous
import functools

import jax
import jax.numpy as jnp
from jax import lax
from jax.experimental import pallas as pl
from jax.experimental.pallas import tpu as pltpu
from jax.experimental.pallas import tpu_sc as plsc

_B, _L, _V, _D = 1024, 200, 1_000_000, 64
_DP = 128
_N = _B * _L
_C = 128
_NW = 32
_RPW = _N // _NW
_CPW = _RPW // _C
_BV = 32768
_NBUF = 4


def _transpose_pad(table_t):
    grid = (pl.cdiv(_V, _BV),)

    def body(in_ref, out_ref):
        out_ref[:, :_D] = in_ref[...].T

    return pl.pallas_call(
        body,
        grid=grid,
        in_specs=[pl.BlockSpec((_D, _BV), lambda i: (0, i))],
        out_specs=pl.BlockSpec((_BV, _DP), lambda i: (i, 0)),
        out_shape=jax.ShapeDtypeStruct((_V, _DP), jnp.float32),
    )(table_t)


def _make_gather():
    mesh = plsc.VectorSubcoreMesh(core_axis_name="c", subcore_axis_name="s")

    @functools.partial(
        pl.kernel,
        mesh=mesh,
        out_type=jax.ShapeDtypeStruct((_N, _DP), jnp.float32),
        scratch_types=[
            pltpu.VMEM((_RPW,), jnp.int32),
            pltpu.VMEM((_NBUF, _C, _DP), jnp.float32),
            pltpu.SemaphoreType.DMA((_NBUF,)),
            pltpu.SemaphoreType.DMA((_NBUF,)),
        ],
    )
    def gather_kernel(idx_hbm, table_hbm, out_hbm, idx_v, rows_v, gsems, osems):
        wid = lax.axis_index("s") * 2 + lax.axis_index("c")
        rbase = wid * _RPW
        pltpu.sync_copy(idx_hbm.at[pl.ds(rbase, _RPW)], idx_v)

        def fire(j):
            b = j % _NBUF
            idx_slice = idx_v.at[pl.ds(j * _C, _C)]
            pltpu.async_copy(table_hbm.at[idx_slice], rows_v.at[b], gsems.at[b])

        for j in range(_NBUF - 1):
            fire(j)

        def body(j, carry):
            b = j % _NBUF
            nb = (j + _NBUF - 1) % _NBUF

            @pl.when((j >= 1) & (j + _NBUF - 1 < _CPW))
            def _():
                pltpu.make_async_copy(
                    rows_v.at[nb], out_hbm.at[pl.ds(rbase, _C)], osems.at[nb]
                ).wait()

            @pl.when(j + _NBUF - 1 < _CPW)
            def _():
                fire(j + _NBUF - 1)

            pltpu.make_async_copy(
                table_hbm.at[idx_v.at[pl.ds(0, _C)]], rows_v.at[b], gsems.at[b]
            ).wait()
            pltpu.async_copy(
                rows_v.at[b], out_hbm.at[pl.ds(rbase + j * _C, _C)], osems.at[b]
            )
            return carry

        lax.fori_loop(0, _CPW, body, 0)
        for b in range(_NBUF):
            pltpu.make_async_copy(
                rows_v.at[b], out_hbm.at[pl.ds(rbase, _C)], osems.at[b]
            ).wait()

    return gather_kernel


_gather = _make_gather()


def kernel(sequences, offsets, table):
    del offsets
    idx = sequences.reshape(_N).astype(jnp.int32)
    table_p = _transpose_pad(table.T)
    out = _gather(idx, table_p)
    return out[:, :_D].reshape(_B, _L, _D)


# speedup vs baseline: 1.2731x; 1.0025x over previous
import functools

import jax
import jax.numpy as jnp
from jax import lax
from jax.experimental import pallas as pl
from jax.experimental.pallas import tpu as pltpu
from jax.experimental.pallas import tpu_sc as plsc

_B, _L, _V, _D = 1024, 200, 1_000_000, 64
_DP = 128
_N = _B * _L
_C = 128
_NW = 32
_RPW = _N // _NW
_CPW = _RPW // _C
_BV = 32768
_NBUF = 6


def _transpose_pad(table_t):
    grid = (pl.cdiv(_V, _BV),)

    def body(in_ref, out_ref):
        out_ref[:, :_D] = in_ref[...].T

    return pl.pallas_call(
        body,
        grid=grid,
        in_specs=[pl.BlockSpec((_D, _BV), lambda i: (0, i))],
        out_specs=pl.BlockSpec((_BV, _DP), lambda i: (i, 0)),
        out_shape=jax.ShapeDtypeStruct((_V, _DP), jnp.float32),
    )(table_t)


def _make_gather():
    mesh = plsc.VectorSubcoreMesh(core_axis_name="c", subcore_axis_name="s")

    @functools.partial(
        pl.kernel,
        mesh=mesh,
        out_type=jax.ShapeDtypeStruct((_N, _DP), jnp.float32),
        scratch_types=[
            pltpu.VMEM((_RPW,), jnp.int32),
            pltpu.VMEM((_NBUF, _C, _DP), jnp.float32),
            pltpu.SemaphoreType.DMA((_NBUF,)),
            pltpu.SemaphoreType.DMA((_NBUF,)),
        ],
    )
    def gather_kernel(idx_hbm, table_hbm, out_hbm, idx_v, rows_v, gsems, osems):
        wid = lax.axis_index("s") * 2 + lax.axis_index("c")
        rbase = wid * _RPW
        pltpu.sync_copy(idx_hbm.at[pl.ds(rbase, _RPW)], idx_v)

        def fire(j):
            b = j % _NBUF
            idx_slice = idx_v.at[pl.ds(j * _C, _C)]
            pltpu.async_copy(table_hbm.at[idx_slice], rows_v.at[b], gsems.at[b])

        for j in range(_NBUF - 1):
            fire(j)

        def body(j, carry):
            b = j % _NBUF
            nb = (j + _NBUF - 1) % _NBUF

            @pl.when((j >= 1) & (j + _NBUF - 1 < _CPW))
            def _():
                pltpu.make_async_copy(
                    rows_v.at[nb], out_hbm.at[pl.ds(rbase, _C)], osems.at[nb]
                ).wait()

            @pl.when(j + _NBUF - 1 < _CPW)
            def _():
                fire(j + _NBUF - 1)

            pltpu.make_async_copy(
                table_hbm.at[idx_v.at[pl.ds(0, _C)]], rows_v.at[b], gsems.at[b]
            ).wait()
            pltpu.async_copy(
                rows_v.at[b], out_hbm.at[pl.ds(rbase + j * _C, _C)], osems.at[b]
            )
            return carry

        lax.fori_loop(0, _CPW, body, 0)
        for b in range(_NBUF):
            pltpu.make_async_copy(
                rows_v.at[b], out_hbm.at[pl.ds(rbase, _C)], osems.at[b]
            ).wait()

    return gather_kernel


_gather = _make_gather()


def kernel(sequences, offsets, table):
    del offsets
    idx = sequences.reshape(_N).astype(jnp.int32)
    table_p = _transpose_pad(table.T)
    out = _gather(idx, table_p)
    return out[:, :_D].reshape(_B, _L, _D)
